# Initial kernel scaffold; baseline (speedup 1.0000x reference)
import functools

import jax
import jax.numpy as jnp
from jax import lax
from jax.experimental import pallas as pl
from jax.experimental.pallas import tpu as pltpu
from jax.experimental.pallas import tpu_sc as plsc

NC = 2
NS = 16
NW = NC * NS
D = 64
GK = 128
GPB = 4
BLK = GK * GPB


@functools.cache
def _make_kernel(B):
    b_per_w = B // NW
    n_blocks = b_per_w // BLK
    mesh = plsc.VectorSubcoreMesh(core_axis_name="c", subcore_axis_name="s")

    @functools.partial(
        pl.kernel,
        out_type=jax.ShapeDtypeStruct((B, D), jnp.float32),
        mesh=mesh,
        scratch_types=[
            pltpu.VMEM((b_per_w,), jnp.int32),
            pltpu.VMEM((BLK, D), jnp.float32),
            pltpu.SemaphoreType.DMA,
        ],
    )
    def gather_kernel(idx_hbm, table_hbm, out_hbm, idx_v, rows_v, sem):
        wid = lax.axis_index("s") * NC + lax.axis_index("c")
        base = wid * b_per_w
        pltpu.sync_copy(idx_hbm.at[pl.ds(base, b_per_w)], idx_v)

        def block(i, carry):
            copies = []
            for b in range(GPB):
                copies.append(pltpu.async_copy(
                    table_hbm.at[idx_v.at[pl.ds(i * BLK + b * GK, GK)]],
                    rows_v.at[pl.ds(b * GK, GK)],
                    sem))
            for c in copies:
                c.wait()
            pltpu.sync_copy(rows_v, out_hbm.at[pl.ds(base + i * BLK, BLK)])
            return carry

        lax.fori_loop(0, n_blocks, block, 0)

    return gather_kernel


@jax.jit
def kernel(x, pe):
    idx = x.reshape(-1).astype(jnp.int32)
    out = _make_kernel(idx.shape[0])(idx, pe)
    return out.reshape(x.shape[0], x.shape[1], pe.shape[1])


# speedup vs baseline: 3.9772x; 3.9772x over previous
import functools

import jax
import jax.numpy as jnp
from jax import lax
from jax.experimental import pallas as pl
from jax.experimental.pallas import tpu as pltpu
from jax.experimental.pallas import tpu_sc as plsc

NC = 2
NS = 16
NW = NC * NS
D = 64
GK = 128
GPB = 4
BLK = GK * GPB


@functools.cache
def _make_kernel(B):
    b_per_w = B // NW
    n_blocks = b_per_w // BLK
    mesh = plsc.VectorSubcoreMesh(core_axis_name="c", subcore_axis_name="s")

    @functools.partial(
        pl.kernel,
        out_type=jax.ShapeDtypeStruct((B, D), jnp.float32),
        mesh=mesh,
        scratch_types=[
            pltpu.VMEM((b_per_w,), jnp.int32),
            pltpu.VMEM((BLK, D), jnp.float32),
            pltpu.SemaphoreType.DMA,
        ],
        compiler_params=pltpu.CompilerParams(use_tc_tiling_on_sc=False),
    )
    def gather_kernel(idx_hbm, table_hbm, out_hbm, idx_v, rows_v, sem):
        wid = lax.axis_index("s") * NC + lax.axis_index("c")
        base = wid * b_per_w
        pltpu.sync_copy(idx_hbm.at[pl.ds(base, b_per_w)], idx_v)

        def block(i, carry):
            copies = []
            for b in range(GPB):
                copies.append(pltpu.async_copy(
                    table_hbm.at[idx_v.at[pl.ds(i * BLK + b * GK, GK)]],
                    rows_v.at[pl.ds(b * GK, GK)],
                    sem))
            for c in copies:
                c.wait()
            pltpu.sync_copy(rows_v, out_hbm.at[pl.ds(base + i * BLK, BLK)])
            return carry

        lax.fori_loop(0, n_blocks, block, 0)

    return gather_kernel


@jax.jit
def kernel(x, pe):
    idx = x.reshape(-1).astype(jnp.int32)
    out = _make_kernel(idx.shape[0])(idx, pe)
    return out.reshape(x.shape[0], x.shape[1], pe.shape[1])


# speedup vs baseline: 3.9784x; 1.0003x over previous
import functools

import jax
import jax.numpy as jnp
from jax import lax
from jax.experimental import pallas as pl
from jax.experimental.pallas import tpu as pltpu
from jax.experimental.pallas import tpu_sc as plsc

NC = 2
NS = 16
NW = NC * NS
D = 64
GK = 512
GPB = 1
BLK = GK * GPB


@functools.cache
def _make_kernel(B):
    b_per_w = B // NW
    n_blocks = b_per_w // BLK
    mesh = plsc.VectorSubcoreMesh(core_axis_name="c", subcore_axis_name="s")

    @functools.partial(
        pl.kernel,
        out_type=jax.ShapeDtypeStruct((B, D), jnp.float32),
        mesh=mesh,
        scratch_types=[
            pltpu.VMEM((b_per_w,), jnp.int32),
            pltpu.VMEM((BLK, D), jnp.float32),
            pltpu.SemaphoreType.DMA,
        ],
        compiler_params=pltpu.CompilerParams(use_tc_tiling_on_sc=False),
    )
    def gather_kernel(idx_hbm, table_hbm, out_hbm, idx_v, rows_v, sem):
        wid = lax.axis_index("s") * NC + lax.axis_index("c")
        base = wid * b_per_w
        pltpu.sync_copy(idx_hbm.at[pl.ds(base, b_per_w)], idx_v)

        def block(i, carry):
            copies = []
            for b in range(GPB):
                copies.append(pltpu.async_copy(
                    table_hbm.at[idx_v.at[pl.ds(i * BLK + b * GK, GK)]],
                    rows_v.at[pl.ds(b * GK, GK)],
                    sem))
            for c in copies:
                c.wait()
            pltpu.sync_copy(rows_v, out_hbm.at[pl.ds(base + i * BLK, BLK)])
            return carry

        lax.fori_loop(0, n_blocks, block, 0)

    return gather_kernel


@jax.jit
def kernel(x, pe):
    idx = x.reshape(-1).astype(jnp.int32)
    out = _make_kernel(idx.shape[0])(idx, pe)
    return out.reshape(x.shape[0], x.shape[1], pe.shape[1])


# speedup vs baseline: 4.0026x; 1.0061x over previous
import functools

import jax
import jax.numpy as jnp
from jax import lax
from jax.experimental import pallas as pl
from jax.experimental.pallas import tpu as pltpu
from jax.experimental.pallas import tpu_sc as plsc

NC = 2
NS = 16
NW = NC * NS
D = 64
BLK = 256
NBUF = 4


@functools.cache
def _make_kernel(B):
    b_per_w = B // NW
    n_blocks = b_per_w // BLK
    assert n_blocks % NBUF == 0 and n_blocks >= 2 * NBUF
    mesh = plsc.VectorSubcoreMesh(core_axis_name="c", subcore_axis_name="s")

    @functools.partial(
        pl.kernel,
        out_type=jax.ShapeDtypeStruct((B, D), jnp.float32),
        mesh=mesh,
        scratch_types=[
            pltpu.VMEM((b_per_w,), jnp.int32),
            pltpu.VMEM((NBUF, BLK, D), jnp.float32),
            pltpu.SemaphoreType.DMA,
            pltpu.SemaphoreType.DMA,
        ],
        compiler_params=pltpu.CompilerParams(use_tc_tiling_on_sc=False),
    )
    def gather_kernel(idx_hbm, table_hbm, out_hbm, idx_v, rows_v, gsem, osem):
        wid = lax.axis_index("s") * NC + lax.axis_index("c")
        base = wid * b_per_w
        pltpu.sync_copy(idx_hbm.at[pl.ds(base, b_per_w)], idx_v)

        def fire_g(blk, b):
            pltpu.async_copy(
                table_hbm.at[idx_v.at[pl.ds(blk * BLK, BLK)]],
                rows_v.at[b], gsem)

        def drain_g(b):
            pltpu.make_async_copy(
                table_hbm.at[idx_v.at[pl.ds(0, BLK)]],
                rows_v.at[b], gsem).wait()

        def fire_o(blk, b):
            pltpu.async_copy(
                rows_v.at[b], out_hbm.at[pl.ds(base + blk * BLK, BLK)], osem)

        def wait_o(b):
            pltpu.make_async_copy(
                rows_v.at[b], out_hbm.at[pl.ds(base, BLK)], osem).wait()

        fire_g(0, 0)
        fire_g(1, 1)
        drain_g(0); fire_o(0, 0); fire_g(2, 2)
        drain_g(1); fire_o(1, 1); fire_g(3, 3)

        def steady(i, carry):
            for j in range(NBUF):
                b = (2 + j) % NBUF
                blk = 2 + i * NBUF + j
                wait_o((b + 2) % NBUF)
                fire_g(blk + 2, (b + 2) % NBUF)
                drain_g(b)
                fire_o(blk, b)
            return carry

        lax.fori_loop(0, (n_blocks - 4) // NBUF, steady, 0)

        for blk in (n_blocks - 2, n_blocks - 1):
            b = blk % NBUF
            drain_g(b)
            fire_o(blk, b)
        for _ in range(NBUF):
            wait_o(0)

    return gather_kernel


@jax.jit
def kernel(x, pe):
    idx = x.reshape(-1).astype(jnp.int32)
    out = _make_kernel(idx.shape[0])(idx, pe)
    return out.reshape(x.shape[0], x.shape[1], pe.shape[1])


# speedup vs baseline: 4.0458x; 1.0108x over previous
import functools

import jax
import jax.numpy as jnp
from jax import lax
from jax.experimental import pallas as pl
from jax.experimental.pallas import tpu as pltpu
from jax.experimental.pallas import tpu_sc as plsc

NC = 2
NS = 16
NW = NC * NS
D = 64
L = 16
T = 200
BB = 128


@functools.cache
def _make_kernel():
    mesh = plsc.VectorSubcoreMesh(core_axis_name="c", subcore_axis_name="s")

    @functools.partial(
        pl.kernel,
        out_type=jax.ShapeDtypeStruct((T, D // 8, NW, 8 * BB), jnp.float32),
        mesh=mesh,
        scratch_types=[
            pltpu.VMEM((BB * T,), jnp.int32),
            pltpu.VMEM((2, BB), jnp.int32),
            pltpu.VMEM((2, BB, D), jnp.float32),
            pltpu.VMEM((2, D // 8, 8 * BB), jnp.float32),
            pltpu.SemaphoreType.DMA,
            pltpu.SemaphoreType.DMA,
        ],
        compiler_params=pltpu.CompilerParams(use_tc_tiling_on_sc=False,
                                             needs_layout_passes=False),
    )
    def gather_kernel(idx_hbm, table_hbm, out_hbm,
                      idx_v, col_v, rows_v, til_v, gsem, osem):
        wid = lax.axis_index("s") * NC + lax.axis_index("c")
        pltpu.sync_copy(idx_hbm.at[pl.ds(wid * BB * T, BB * T)], idx_v)

        iota = lax.iota(jnp.int32, L)
        iota_t = iota * T
        cg_vecs = [(iota >> 3) + 2 * cg16 for cg16 in range(D // L)]
        in_base = (iota & 7) * BB

        def build_col(t, cb):
            for j in range(BB // L):
                v = plsc.load_gather(idx_v, [iota_t + (j * L * T + t)])
                col_v[cb, pl.ds(j * L, L)] = v

        def fire_g(cb, rb):
            pltpu.async_copy(table_hbm.at[col_v.at[cb]], rows_v.at[rb], gsem)

        def drain_g(rb):
            pltpu.make_async_copy(
                table_hbm.at[col_v.at[0]], rows_v.at[rb], gsem).wait()

        def transpose(rb, tb):
            til = til_v.at[tb]

            @plsc.parallel_loop(0, BB, step=1, unroll=8)
            def _(bl):
                in_vec = in_base + bl
                for cg16 in range(D // L):
                    v = rows_v[rb, bl, pl.ds(cg16 * L, L)]
                    plsc.store_scatter(til, [cg_vecs[cg16], in_vec], v)

        def fire_o(t, tb):
            pltpu.async_copy(til_v.at[tb], out_hbm.at[t, :, wid], osem)

        def wait_o():
            pltpu.make_async_copy(
                til_v.at[0], out_hbm.at[0, :, wid], osem).wait()

        build_col(0, 0)
        fire_g(0, 0)

        def half(t, buf, nbuf):
            drain_g(buf)

            @pl.when(t < T - 1)
            def _():
                build_col(t + 1, nbuf)
                fire_g(nbuf, nbuf)

            @pl.when(t >= 2)
            def _():
                wait_o()

            transpose(buf, buf)
            fire_o(t, buf)

        def body(i, carry):
            t = 2 * i
            half(t, 0, 1)
            half(t + 1, 1, 0)
            return carry

        lax.fori_loop(0, T // 2, body, 0)
        wait_o()
        wait_o()

    return gather_kernel


@jax.jit
def kernel(x, pe):
    idx = x.reshape(-1).astype(jnp.int32)
    a = _make_kernel()(idx, pe)
    a = a.reshape(T, D // 8, NW, 8, BB).transpose(2, 4, 0, 1, 3)
    return a.reshape(x.shape[0], x.shape[1], pe.shape[1])


# speedup vs baseline: 9.4645x; 2.3394x over previous
import functools

import jax
import jax.numpy as jnp
from jax import lax
from jax.experimental import pallas as pl
from jax.experimental.pallas import tpu as pltpu
from jax.experimental.pallas import tpu_sc as plsc

NC = 2
NS = 16
NW = NC * NS
D = 64
L = 16
T = 200
BB = 128


@functools.cache
def _make_kernel():
    mesh = plsc.VectorSubcoreMesh(core_axis_name="c", subcore_axis_name="s")

    @functools.partial(
        pl.kernel,
        out_type=jax.ShapeDtypeStruct((T, D // 8, NW, 8 * BB), jnp.float32),
        mesh=mesh,
        scratch_types=[
            pltpu.VMEM((BB * T,), jnp.int32),
            pltpu.VMEM((2, BB), jnp.int32),
            pltpu.VMEM((2, BB, D), jnp.float32),
            pltpu.VMEM((2, D * BB), jnp.float32),
            pltpu.SemaphoreType.DMA,
            pltpu.SemaphoreType.DMA,
        ],
        compiler_params=pltpu.CompilerParams(use_tc_tiling_on_sc=False,
                                             needs_layout_passes=False),
    )
    def gather_kernel(idx_hbm, table_hbm, out_hbm,
                      idx_v, col_v, rows_v, til_v, gsem, osem):
        wid = lax.axis_index("s") * NC + lax.axis_index("c")
        pltpu.sync_copy(idx_hbm.at[pl.ds(wid * BB * T, BB * T)], idx_v)

        iota = lax.iota(jnp.int32, L)
        iota_t = iota * T
        czero = jnp.zeros((L,), jnp.int32)

        def build_col(t, cb):
            for j in range(BB // L):
                v = plsc.load_gather(idx_v, [iota_t + (j * L * T + t)])
                col_v[cb, pl.ds(j * L, L)] = v

        def fire_g(cb, rb):
            pltpu.async_copy(table_hbm.at[col_v.at[cb]], rows_v.at[rb], gsem)

        def drain_g(rb):
            pltpu.make_async_copy(
                table_hbm.at[col_v.at[0]], rows_v.at[rb], gsem).wait()

        def transpose(rb, tb):
            rows = rows_v.at[rb]
            til = til_v.at[tb]

            @plsc.parallel_loop(0, L, step=1, unroll=2)
            def _(k):
                rot = (iota + k) & (L - 1)
                st_base = (rot << 7) + iota
                for c0 in range(D // L):
                    cvec = rot + (c0 * L)
                    for j in range(BB // L):
                        v = plsc.load_gather(rows, [iota + j * L, cvec])
                        plsc.store_scatter(
                            til, [st_base + (c0 * L * BB + j * L)], v)

        def fire_o(t, tb):
            for cg in range(D // 8):
                pltpu.async_copy(til_v.at[tb, pl.ds(cg * 8 * BB, 8 * BB)],
                                 out_hbm.at[t, cg, wid], osem)

        def wait_o():
            for cg in range(D // 8):
                pltpu.make_async_copy(
                    til_v.at[0, pl.ds(0, 8 * BB)],
                    out_hbm.at[0, 0, wid], osem).wait()

        build_col(0, 0)
        fire_g(0, 0)

        def half(t, buf, nbuf):
            drain_g(buf)

            @pl.when(t < T - 1)
            def _():
                build_col(t + 1, nbuf)
                fire_g(nbuf, nbuf)

            @pl.when(t >= 2)
            def _():
                wait_o()

            transpose(buf, buf)
            fire_o(t, buf)

        def body(i, carry):
            t = 2 * i
            half(t, 0, 1)
            half(t + 1, 1, 0)
            return carry

        lax.fori_loop(0, T // 2, body, 0)
        wait_o()
        wait_o()

    return gather_kernel


@jax.jit
def kernel(x, pe):
    idx = x.reshape(-1).astype(jnp.int32)
    a = _make_kernel()(idx, pe)
    a = a.reshape(T, D // 8, NW, 8, BB).transpose(2, 4, 0, 1, 3)
    return a.reshape(x.shape[0], x.shape[1], pe.shape[1])


# speedup vs baseline: 9.9105x; 1.0471x over previous
import functools

import jax
import jax.numpy as jnp
from jax import lax
from jax.experimental import pallas as pl
from jax.experimental.pallas import tpu as pltpu
from jax.experimental.pallas import tpu_sc as plsc

NC = 2
NS = 16
NW = NC * NS
D = 64
L = 16
T = 200
BB = 128


@functools.cache
def _make_kernel():
    mesh = plsc.VectorSubcoreMesh(core_axis_name="c", subcore_axis_name="s")

    @functools.partial(
        pl.kernel,
        out_type=jax.ShapeDtypeStruct((T, D // 8, NW, 8 * BB), jnp.float32),
        mesh=mesh,
        scratch_types=[
            pltpu.VMEM((T, BB), jnp.int32),
            pltpu.VMEM((2, BB, D), jnp.float32),
            pltpu.VMEM((2, D * BB), jnp.float32),
            pltpu.SemaphoreType.DMA,
            pltpu.SemaphoreType.DMA,
        ],
        compiler_params=pltpu.CompilerParams(use_tc_tiling_on_sc=False,
                                             needs_layout_passes=False),
    )
    def gather_kernel(idx_hbm, table_hbm, out_hbm,
                      idx_v, rows_v, til_v, gsem, osem):
        wid = lax.axis_index("s") * NC + lax.axis_index("c")
        pltpu.sync_copy(idx_hbm.at[:, pl.ds(wid * BB, BB)], idx_v)

        iota = lax.iota(jnp.int32, L)

        def fire_g(t, rb):
            pltpu.async_copy(table_hbm.at[idx_v.at[t]], rows_v.at[rb], gsem)

        def drain_g(rb):
            pltpu.make_async_copy(
                table_hbm.at[idx_v.at[0]], rows_v.at[rb], gsem).wait()

        def transpose(rb, tb):
            rows = rows_v.at[rb]
            til = til_v.at[tb]

            @plsc.parallel_loop(0, L, step=1, unroll=2)
            def _(k):
                rot = (iota + k) & (L - 1)
                st_base = (rot << 7) + iota
                for c0 in range(D // L):
                    cvec = rot + (c0 * L)
                    for j in range(BB // L):
                        v = plsc.load_gather(rows, [iota + j * L, cvec])
                        plsc.store_scatter(
                            til, [st_base + (c0 * L * BB + j * L)], v)

        def fire_o(t, tb):
            for cg in range(D // 8):
                pltpu.async_copy(til_v.at[tb, pl.ds(cg * 8 * BB, 8 * BB)],
                                 out_hbm.at[t, cg, wid], osem)

        def wait_o():
            for cg in range(D // 8):
                pltpu.make_async_copy(
                    til_v.at[0, pl.ds(0, 8 * BB)],
                    out_hbm.at[0, 0, wid], osem).wait()

        fire_g(0, 0)

        def half(t, buf, nbuf):
            drain_g(buf)

            @pl.when(t < T - 1)
            def _():
                fire_g(t + 1, nbuf)

            @pl.when(t >= 2)
            def _():
                wait_o()

            transpose(buf, buf)
            fire_o(t, buf)

        def body(i, carry):
            t = 2 * i
            half(t, 0, 1)
            half(t + 1, 1, 0)
            return carry

        lax.fori_loop(0, T // 2, body, 0)
        wait_o()
        wait_o()

    return gather_kernel


@jax.jit
def kernel(x, pe):
    idx = x.reshape(x.shape[0], x.shape[1]).astype(jnp.int32).T
    a = _make_kernel()(idx, pe)
    a = a.reshape(T, D // 8, NW, 8, BB).transpose(2, 4, 0, 1, 3)
    return a.reshape(x.shape[0], x.shape[1], pe.shape[1])


# speedup vs baseline: 10.1649x; 1.0257x over previous
import functools

import jax
import jax.numpy as jnp
from jax import lax
from jax.experimental import pallas as pl
from jax.experimental.pallas import tpu as pltpu
from jax.experimental.pallas import tpu_sc as plsc

NC = 2
NS = 16
NW = NC * NS
D = 64
L = 16
T = 200
BB = 128


@functools.cache
def _make_kernel():
    mesh = plsc.VectorSubcoreMesh(core_axis_name="c", subcore_axis_name="s")

    @functools.partial(
        pl.kernel,
        out_type=jax.ShapeDtypeStruct((T, D // 8, NW, 8 * BB), jnp.float32),
        mesh=mesh,
        scratch_types=[
            pltpu.VMEM((T, BB), jnp.int32),
            pltpu.VMEM((4, BB, D), jnp.float32),
            pltpu.VMEM((4, D * BB), jnp.float32),
            pltpu.SemaphoreType.DMA,
            pltpu.SemaphoreType.DMA,
        ],
        compiler_params=pltpu.CompilerParams(use_tc_tiling_on_sc=False,
                                             needs_layout_passes=False),
    )
    def gather_kernel(idx_hbm, table_hbm, out_hbm,
                      idx_v, rows_v, til_v, gsem, osem):
        wid = lax.axis_index("s") * NC + lax.axis_index("c")
        pltpu.sync_copy(idx_hbm.at[:, pl.ds(wid * BB, BB)], idx_v)

        iota = lax.iota(jnp.int32, L)

        def fire_g(t, rb):
            pltpu.async_copy(table_hbm.at[idx_v.at[t]], rows_v.at[rb], gsem)

        def drain_g(rb):
            pltpu.make_async_copy(
                table_hbm.at[idx_v.at[0]], rows_v.at[rb], gsem).wait()

        def transpose(rb, tb):
            rows = rows_v.at[rb]
            til = til_v.at[tb]

            @plsc.parallel_loop(0, L, step=1, unroll=2)
            def _(k):
                rot = (iota + k) & (L - 1)
                st_base = (rot << 7) + iota
                for c0 in range(D // L):
                    cvec = rot + (c0 * L)
                    for j in range(BB // L):
                        v = plsc.load_gather(rows, [iota + j * L, cvec])
                        plsc.store_scatter(
                            til, [st_base + (c0 * L * BB + j * L)], v)

        def fire_o(t, tb):
            for cg in range(D // 8):
                pltpu.async_copy(til_v.at[tb, pl.ds(cg * 8 * BB, 8 * BB)],
                                 out_hbm.at[t, cg, wid], osem)

        def wait_o():
            for cg in range(D // 8):
                pltpu.make_async_copy(
                    til_v.at[0, pl.ds(0, 8 * BB)],
                    out_hbm.at[0, 0, wid], osem).wait()

        fire_g(0, 0)
        fire_g(1, 1)
        fire_g(2, 2)

        def step(t, buf):
            drain_g(buf)

            @pl.when(t < T - 3)
            def _():
                fire_g(t + 3, (buf + 3) % 4)

            @pl.when(t >= 4)
            def _():
                wait_o()

            transpose(buf, buf)
            fire_o(t, buf)

        def body(i, carry):
            t = 4 * i
            for j in range(4):
                step(t + j, j)
            return carry

        lax.fori_loop(0, T // 4, body, 0)
        for _ in range(4):
            wait_o()

    return gather_kernel


@jax.jit
def kernel(x, pe):
    idx = x.reshape(x.shape[0], x.shape[1]).astype(jnp.int32).T
    a = _make_kernel()(idx, pe)
    a = a.reshape(T, D // 8, NW, 8, BB).transpose(2, 4, 0, 1, 3)
    return a.reshape(x.shape[0], x.shape[1], pe.shape[1])
